# Initial kernel scaffold; baseline (speedup 1.0000x reference)
import jax
import jax.numpy as jnp
from jax import lax
from jax.experimental import pallas as pl
from jax.experimental.pallas import tpu as pltpu
from jax.experimental.pallas import tpu_sc as plsc

HIDDEN = 128
NB = HIDDEN // 16
CHUNK = 400


def _body(e0, e1, e2, t0, t1, t2, out,
          idx0, idx1, idx2, acc, tmp1, tmp2, sem0, sem1, sem2):
    nc = 2
    wid = lax.axis_index("s") * nc + lax.axis_index("c")
    e_total = out.shape[0]
    per_tile = e_total // 32
    n_chunks = per_tile // CHUNK
    base = wid * per_tile

    def chunk_body(k, _):
        off = base + k * CHUNK
        pltpu.sync_copy(e0.at[pl.ds(off, CHUNK)], idx0)
        pltpu.sync_copy(e1.at[pl.ds(off, CHUNK)], idx1)
        pltpu.sync_copy(e2.at[pl.ds(off, CHUNK)], idx2)
        c0 = pltpu.async_copy(t0.at[idx0], acc, sem0)
        c1 = pltpu.async_copy(t1.at[idx1], tmp1, sem1)
        c2 = pltpu.async_copy(t2.at[idx2], tmp2, sem2)
        c0.wait()
        c1.wait()
        c2.wait()

        def add_body(i, _):
            for b in range(NB):
                s = pl.ds(b * 16, 16)
                acc[i, s] = acc[i, s] + tmp1[i, s] + tmp2[i, s]
            return 0

        lax.fori_loop(0, CHUNK, add_body, 0, unroll=2)
        pltpu.sync_copy(acc, out.at[pl.ds(off, CHUNK), :])
        return 0

    lax.fori_loop(0, n_chunks, chunk_body, 0)


def kernel(edge_attr, emb0, emb1, emb2):
    e_count = edge_attr.shape[0]
    e_t = edge_attr.T
    e0, e1, e2 = e_t[0], e_t[1], e_t[2]

    mesh = plsc.VectorSubcoreMesh(core_axis_name="c", subcore_axis_name="s")
    run = pl.kernel(
        _body,
        out_type=jax.ShapeDtypeStruct((e_count, HIDDEN), jnp.float32),
        mesh=mesh,
        scratch_types=[
            pltpu.VMEM((CHUNK,), jnp.int32),
            pltpu.VMEM((CHUNK,), jnp.int32),
            pltpu.VMEM((CHUNK,), jnp.int32),
            pltpu.VMEM((CHUNK, HIDDEN), jnp.float32),
            pltpu.VMEM((CHUNK, HIDDEN), jnp.float32),
            pltpu.VMEM((CHUNK, HIDDEN), jnp.float32),
            pltpu.SemaphoreType.DMA,
            pltpu.SemaphoreType.DMA,
            pltpu.SemaphoreType.DMA,
        ],
    )
    return run(e0, e1, e2, emb0, emb1, emb2)


# speedup vs baseline: 2.1790x; 2.1790x over previous
import jax
import jax.numpy as jnp
from jax import lax
from jax.experimental import pallas as pl
from jax.experimental.pallas import tpu as pltpu
from jax.experimental.pallas import tpu_sc as plsc

HIDDEN = 128
NB = HIDDEN // 16
CHUNK = 200


def _body(e0, e1, e2, t0, t1, t2, out,
          idx0, idx1, idx2, acc, tmp1, tmp2, sem0, sem1, sem2):
    nc = 2
    wid = lax.axis_index("s") * nc + lax.axis_index("c")
    e_total = out.shape[0]
    per_tile = e_total // 32
    n_chunks = per_tile // CHUNK
    base = wid * per_tile

    def chunk_body(k, _):
        off = base + k * CHUNK
        pltpu.sync_copy(e0.at[pl.ds(off, CHUNK)], idx0)
        pltpu.sync_copy(e1.at[pl.ds(off, CHUNK)], idx1)
        pltpu.sync_copy(e2.at[pl.ds(off, CHUNK)], idx2)
        c0 = pltpu.async_copy(t0.at[idx0], acc, sem0)
        c1 = pltpu.async_copy(t1.at[idx1], tmp1, sem1)
        c2 = pltpu.async_copy(t2.at[idx2], tmp2, sem2)
        c0.wait()
        c1.wait()
        c2.wait()

        def add_body(i, _):
            for b in range(NB):
                s = pl.ds(b * 16, 16)
                acc[i, s] = acc[i, s] + tmp1[i, s] + tmp2[i, s]
            return 0

        lax.fori_loop(0, CHUNK, add_body, 0, unroll=2)
        pltpu.sync_copy(acc, out.at[pl.ds(off, CHUNK), :])
        return 0

    lax.fori_loop(0, n_chunks, chunk_body, 0)


def kernel(edge_attr, emb0, emb1, emb2):
    e_count = edge_attr.shape[0]
    e_t = edge_attr.T
    e0, e1, e2 = e_t[0], e_t[1], e_t[2]

    mesh = plsc.VectorSubcoreMesh(core_axis_name="c", subcore_axis_name="s")
    run = pl.kernel(
        _body,
        out_type=jax.ShapeDtypeStruct((e_count, HIDDEN), jnp.float32),
        mesh=mesh,
        scratch_types=[
            pltpu.VMEM((CHUNK,), jnp.int32),
            pltpu.VMEM((CHUNK,), jnp.int32),
            pltpu.VMEM((CHUNK,), jnp.int32),
            pltpu.VMEM((CHUNK, HIDDEN), jnp.float32),
            pltpu.VMEM((CHUNK, HIDDEN), jnp.float32),
            pltpu.VMEM((CHUNK, HIDDEN), jnp.float32),
            pltpu.SemaphoreType.DMA,
            pltpu.SemaphoreType.DMA,
            pltpu.SemaphoreType.DMA,
        ],
    )
    return run(e0, e1, e2, emb0, emb1, emb2)


# speedup vs baseline: 7.7474x; 3.5555x over previous
import jax
import jax.numpy as jnp
from jax import lax
from jax.experimental import pallas as pl
from jax.experimental.pallas import tpu as pltpu
from jax.experimental.pallas import tpu_sc as plsc

HIDDEN = 128
NB = HIDDEN // 16
CHUNK = 80
NW = 32


def _body(e0, e1, e2, t0, t1, t2, out,
          t0_sp, t1_sp, t2_sp, idx0, idx1, idx2,
          rows, outbuf, sem_g, sem_o):
    nc = 2
    cid = lax.axis_index("c")
    sid = lax.axis_index("s")
    wid = sid * nc + cid
    e_total = out.shape[0]
    per_tile = e_total // NW
    n_chunks = per_tile // CHUNK
    base = wid * per_tile

    @pl.when(sid == 0)
    def _stage():
        pltpu.sync_copy(t0, t0_sp)
        pltpu.sync_copy(t1, t1_sp)
        pltpu.sync_copy(t2, t2_sp)

    pltpu.sync_copy(e0.at[pl.ds(base, per_tile)], idx0)
    pltpu.sync_copy(e1.at[pl.ds(base, per_tile)], idx1)
    pltpu.sync_copy(e2.at[pl.ds(base, per_tile)], idx2)
    plsc.subcore_barrier()

    tables = (t0_sp, t1_sp, t2_sp)
    idxs = (idx0, idx1, idx2)

    def issue_gathers(k, p):
        o = k * CHUNK
        for j in range(3):
            pltpu.async_copy(
                tables[j].at[idxs[j].at[pl.ds(o, CHUNK)]], rows[p][j], sem_g[p])

    def wait_gathers(p):
        for j in range(3):
            pltpu.make_async_copy(
                tables[j].at[idxs[j].at[pl.ds(0, CHUNK)]], rows[p][j],
                sem_g[p]).wait()

    def wait_out(p):
        pltpu.make_async_copy(
            outbuf[p], out.at[pl.ds(base, CHUNK), :], sem_o[p]).wait()

    issue_gathers(0, 0)
    issue_gathers(1, 1)

    def chunk_body(k, _):
        p = lax.rem(k, 2)

        def run(p):
            wait_gathers(p)

            @pl.when(k >= 2)
            def _():
                wait_out(p)

            r0, r1, r2 = rows[p]
            ob = outbuf[p]

            @plsc.parallel_loop(0, CHUNK, unroll=2)
            def _(i):
                for b in range(NB):
                    s = pl.ds(b * 16, 16)
                    ob[i, s] = r0[i, s] + r1[i, s] + r2[i, s]

            pltpu.async_copy(
                ob, out.at[pl.ds(base + k * CHUNK, CHUNK), :], sem_o[p])

            @pl.when(k + 2 < n_chunks)
            def _():
                issue_gathers(k + 2, p)

        @pl.when(p == 0)
        def _():
            run(0)

        @pl.when(p == 1)
        def _():
            run(1)

        return 0

    lax.fori_loop(0, n_chunks, chunk_body, 0)
    wait_out(0)
    wait_out(1)


def kernel(edge_attr, emb0, emb1, emb2):
    e_count = edge_attr.shape[0]
    e_t = edge_attr.T
    e0, e1, e2 = e_t[0], e_t[1], e_t[2]
    per_tile = e_count // NW

    mesh = plsc.VectorSubcoreMesh(core_axis_name="c", subcore_axis_name="s")
    run = pl.kernel(
        _body,
        out_type=jax.ShapeDtypeStruct((e_count, HIDDEN), jnp.float32),
        mesh=mesh,
        scratch_types=[
            pltpu.VMEM_SHARED((100, HIDDEN), jnp.float32),
            pltpu.VMEM_SHARED((100, HIDDEN), jnp.float32),
            pltpu.VMEM_SHARED((100, HIDDEN), jnp.float32),
            pltpu.VMEM((per_tile,), jnp.int32),
            pltpu.VMEM((per_tile,), jnp.int32),
            pltpu.VMEM((per_tile,), jnp.int32),
            [[pltpu.VMEM((CHUNK, HIDDEN), jnp.float32) for _ in range(3)]
             for _ in range(2)],
            [pltpu.VMEM((CHUNK, HIDDEN), jnp.float32) for _ in range(2)],
            [pltpu.SemaphoreType.DMA for _ in range(2)],
            [pltpu.SemaphoreType.DMA for _ in range(2)],
        ],
    )
    return run(e0, e1, e2, emb0, emb1, emb2)
